# baseline (device time: 20998 ns/iter reference)
import jax
import jax.numpy as jnp
from jax import lax
from jax.experimental import pallas as pl
from jax.experimental.pallas import tpu as pltpu

Y = 4
BM = 512


def kernel(x, dy, gamma):
    del gamma
    m, d = x.shape

    def body(x_ref, dy_ref, out_ref, self_ref, comm_ref, send_sems, recv_sems):
        step = pl.program_id(0)
        nsteps = pl.num_programs(0)
        my_x = lax.axis_index("x")
        my_y = lax.axis_index("y")
        my_z = lax.axis_index("z")

        @pl.when(step == 0)
        def _():
            barrier_sem = pltpu.get_barrier_semaphore()
            for k in range(1, Y):
                pl.semaphore_signal(
                    barrier_sem,
                    inc=1,
                    device_id=(my_x, lax.rem(my_y + k, Y), my_z),
                    device_id_type=pl.DeviceIdType.MESH,
                )
            pl.semaphore_wait(barrier_sem, Y - 1)

        xv = x_ref[:, :]
        dyv = dy_ref[:, :]
        bm, dd = xv.shape
        s1 = jnp.sum(xv, axis=1, keepdims=True)
        s2 = jnp.sum(xv * xv, axis=1, keepdims=True)
        mu = s1 * (1.0 / dd)
        var = s2 * (1.0 / dd) - mu * mu
        rstd = lax.rsqrt(var + 1e-5)
        a_t = jnp.reshape(rstd, (1, bm))
        b_t = jnp.reshape(-mu * rstd, (1, bm))
        w = jnp.concatenate([b_t, jnp.ones((1, bm), jnp.float32)], axis=0)
        dn = (((1,), (0,)), ((), ()))
        d1 = lax.dot_general(a_t, dyv * xv, dn, preferred_element_type=jnp.float32)
        d2 = lax.dot_general(w, dyv, dn, preferred_element_type=jnp.float32)
        dgamma = d1[0] + d2[0]
        dbeta = d2[1]

        @pl.when(step == 0)
        def _():
            self_ref[0, :] = dgamma
            self_ref[1, :] = dbeta

        @pl.when(step != 0)
        def _():
            self_ref[0, :] = self_ref[0, :] + dgamma
            self_ref[1, :] = self_ref[1, :] + dbeta

        @pl.when(step == nsteps - 1)
        def _():
            sends = []
            for k in range(1, Y):
                tgt_y = lax.rem(my_y + k, Y)
                rdma = pltpu.make_async_remote_copy(
                    src_ref=self_ref,
                    dst_ref=comm_ref.at[Y - 1 - k],
                    send_sem=send_sems.at[k - 1],
                    recv_sem=recv_sems.at[Y - 1 - k],
                    device_id=(my_x, tgt_y, my_z),
                    device_id_type=pl.DeviceIdType.MESH,
                )
                rdma.start()
                sends.append(rdma)

            for r in sends:
                r.wait_recv()
            acc = self_ref[:, :]
            for s in range(Y - 1):
                acc = acc + comm_ref[s, :, :]
            out_ref[:, :] = acc
            for r in sends:
                r.wait_send()

    return pl.pallas_call(
        body,
        grid=(m // BM,),
        out_shape=jax.ShapeDtypeStruct((2, d), jnp.float32),
        in_specs=[
            pl.BlockSpec((BM, d), lambda i: (i, 0)),
            pl.BlockSpec((BM, d), lambda i: (i, 0)),
        ],
        out_specs=pl.BlockSpec((2, d), lambda i: (0, 0)),
        scratch_shapes=[
            pltpu.VMEM((2, d), jnp.float32),
            pltpu.VMEM((Y - 1, 2, d), jnp.float32),
            pltpu.SemaphoreType.DMA((Y - 1,)),
            pltpu.SemaphoreType.DMA((Y - 1,)),
        ],
        compiler_params=pltpu.CompilerParams(collective_id=0),
    )(x, dy)
